# baseline (speedup 1.0000x reference)
import functools

import jax
import jax.numpy as jnp
from jax import lax
from jax.experimental import pallas as pl
from jax.experimental.pallas import tpu as pltpu
from jax.experimental.pallas import tpu_sc as plsc

B = 16
L = 2048
N = 64
CHUNK = 64
HALF = L // 2
NCHUNK = HALF // CHUNK
NORM_EVERY = 8


def _partition_body(emf_ref, emb_ref, trans_ref, head_ref, tail_ref, len_ref,
                    out_ref, a_ref, b_ref, m_ref, mb_ref, pexp_ref, pm_ref,
                    expT_ref, expTT_ref, etail_ref):
    c = pl.program_id(0)
    lens = len_ref[...]

    @pl.when(c == 0)
    def _init():
        tr = trans_ref[...]
        expT_ref[...] = jnp.exp(tr).astype(jnp.bfloat16)
        expTT_ref[...] = jnp.exp(tr.T).astype(jnp.bfloat16)
        et = jnp.exp(tail_ref[...])
        etail_ref[...] = et
        a_ref[...] = jnp.exp(emf_ref[0] + head_ref[...])
        b_ref[...] = jnp.broadcast_to(et, (B, N))
        m_ref[...] = jnp.zeros_like(m_ref)
        mb_ref[...] = jnp.zeros_like(mb_ref)
        pexp_ref[...] = jnp.ones_like(pexp_ref)
        pm_ref[...] = jnp.zeros_like(pm_ref)

    expT = expT_ref[...]
    expTT = expTT_ref[...]
    etail = etail_ref[...]
    a = a_ref[...]
    beta = b_ref[...]
    m = m_ref[...]
    mb = mb_ref[...]
    pexp = pexp_ref[...]
    pm = pm_ref[...]

    for s in range(CHUNK):
        i = c * CHUNK + s
        upd = jax.lax.dot_general(
            a.astype(jnp.bfloat16), expT, (((1,), (0,)), ((), ())),
            preferred_element_type=jnp.float32)
        upd = upd * jnp.exp(emf_ref[s])
        if s == 0:
            a = jnp.where(i == 0, a, upd)
        else:
            a = upd
        dotv = jnp.sum(a * etail, axis=1, keepdims=True)
        pred = lens == (i + 1)
        pexp = jnp.where(pred, dotv, pexp)
        pm = jnp.where(pred, m, pm)
        x = beta * jnp.exp(emb_ref[CHUNK - 1 - s])
        beta = jax.lax.dot_general(
            x.astype(jnp.bfloat16), expTT, (((1,), (0,)), ((), ())),
            preferred_element_type=jnp.float32)
        predb = lens == (L - 1 - i)
        beta = jnp.where(predb, etail, beta)
        mb = jnp.where(predb, 0.0, mb)
        if s % NORM_EVERY == NORM_EVERY - 1:
            sa = jnp.max(a, axis=1, keepdims=True)
            a = a * (1.0 / sa)
            m = m + jnp.log(sa)
            sb = jnp.max(beta, axis=1, keepdims=True)
            beta = beta * (1.0 / sb)
            mb = mb + jnp.log(sb)

    a_ref[...] = a
    b_ref[...] = beta
    m_ref[...] = m
    mb_ref[...] = mb
    pexp_ref[...] = pexp
    pm_ref[...] = pm

    @pl.when(c == NCHUNK - 1)
    def _fin():
        bridge = jnp.sum(a_ref[...] * b_ref[...], axis=1, keepdims=True)
        plong = m_ref[...] + mb_ref[...] + jnp.log(bridge)
        pshort = pm_ref[...] + jnp.log(pexp_ref[...])
        out_ref[...] = jnp.where(lens > HALF, plong, pshort)


@functools.partial(jax.jit, static_argnames=("interpret",))
def _partitions_tc(em_t, trans, head, tail, lengths, interpret=False):
    nblk = L // CHUNK
    return pl.pallas_call(
        _partition_body,
        grid=(NCHUNK,),
        in_specs=[
            pl.BlockSpec((CHUNK, B, N), lambda c: (c, 0, 0)),
            pl.BlockSpec((CHUNK, B, N), lambda c: (nblk - 1 - c, 0, 0)),
            pl.BlockSpec((N, N), lambda c: (0, 0)),
            pl.BlockSpec((1, N), lambda c: (0, 0)),
            pl.BlockSpec((1, N), lambda c: (0, 0)),
            pl.BlockSpec((B, 1), lambda c: (0, 0)),
        ],
        out_specs=pl.BlockSpec((B, 1), lambda c: (0, 0)),
        out_shape=jax.ShapeDtypeStruct((B, 1), jnp.float32),
        scratch_shapes=[
            pltpu.VMEM((B, N), jnp.float32),
            pltpu.VMEM((B, N), jnp.float32),
            pltpu.VMEM((B, 1), jnp.float32),
            pltpu.VMEM((B, 1), jnp.float32),
            pltpu.VMEM((B, 1), jnp.float32),
            pltpu.VMEM((B, 1), jnp.float32),
            pltpu.VMEM((N, N), jnp.bfloat16),
            pltpu.VMEM((N, N), jnp.bfloat16),
            pltpu.VMEM((1, N), jnp.float32),
        ],
        interpret=interpret,
    )(em_t, em_t, trans, head, tail, lengths)


HALF_L = L // 2
_SC_MESH = plsc.VectorSubcoreMesh(core_axis_name="c", subcore_axis_name="s")


def _scores_sc_body(em_hbm, tags_hbm, lens_hbm, trans_hbm, head_hbm, tail_hbm,
                    out_hbm, em_v, tags_v, trans_v, ht_v, lens_v, out_v):
    wid = lax.axis_index("s") * 2 + lax.axis_index("c")
    b = wid // 2
    half = wid % 2
    t0 = half * HALF_L
    pltpu.sync_copy(em_hbm.at[b, pl.ds(t0 * N, HALF_L * N)], em_v)
    pltpu.sync_copy(tags_hbm.at[b], tags_v)
    pltpu.sync_copy(trans_hbm, trans_v)
    pltpu.sync_copy(head_hbm, ht_v.at[pl.ds(0, N)])
    pltpu.sync_copy(tail_hbm, ht_v.at[pl.ds(N, N)])
    pltpu.sync_copy(lens_hbm, lens_v.at[pl.ds(0, B)])

    lane = lax.iota(jnp.int32, 16)
    lenb = plsc.bitcast(
        plsc.load_gather(lens_v, [jnp.full((16,), b, jnp.int32)]), jnp.int32)
    zero = jnp.zeros((16,), jnp.float32)

    def body(i, carry):
        acc_em, acc_tr = carry
        tl = i * 16 + lane
        tg = tl + t0
        tagv = plsc.bitcast(plsc.load_gather(tags_v, [tg]), jnp.int32)
        prev = plsc.bitcast(
            plsc.load_gather(tags_v, [jnp.maximum(tg - 1, 0)]), jnp.int32)
        emv = plsc.load_gather(em_v, [tl * N + tagv])
        trv = plsc.load_gather(trans_v, [prev * N + tagv])
        m_em = tg < lenb
        m_tr = (tg >= 1) & (tg < lenb)
        acc_em = acc_em + jnp.where(m_em, emv, 0.0)
        acc_tr = acc_tr + jnp.where(m_tr, trv, 0.0)
        return acc_em, acc_tr

    acc_em, acc_tr = lax.fori_loop(0, HALF_L // 16, body, (zero, zero))
    total = acc_em + acc_tr

    halfv = jnp.full((16,), half, jnp.int32)
    lane0 = lane == 0
    tag0 = plsc.bitcast(
        plsc.load_gather(tags_v, [jnp.zeros((16,), jnp.int32)]), jnp.int32)
    headv = plsc.load_gather(ht_v, [tag0])
    total = total + jnp.where(lane0 & (halfv == 0), headv, 0.0)
    ttail = lenb - 1
    cover = (ttail >= t0) & (ttail < t0 + HALF_L)
    tagt = plsc.bitcast(plsc.load_gather(tags_v, [ttail]), jnp.int32)
    tailv = plsc.load_gather(ht_v, [N + tagt])
    total = total + jnp.where(lane0 & cover, tailv, 0.0)

    out_v[...] = total
    pltpu.sync_copy(out_v, out_hbm.at[b, half])


@jax.jit
def _scores_sc(em, tg, lens, trans, head, tail):
    k = pl.kernel(
        _scores_sc_body,
        out_type=jax.ShapeDtypeStruct((B, 2, 16), jnp.float32),
        mesh=_SC_MESH,
        compiler_params=pltpu.CompilerParams(needs_layout_passes=False),
        scratch_types=[
            pltpu.VMEM((HALF_L * N,), jnp.float32),
            pltpu.VMEM((L,), jnp.float32),
            pltpu.VMEM((N * N,), jnp.float32),
            pltpu.VMEM((2 * N,), jnp.float32),
            pltpu.VMEM((128,), jnp.float32),
            pltpu.VMEM((16,), jnp.float32),
        ],
    )
    return k(em, tg, lens, trans, head, tail)


def _scores_jax(emissions, tags, lengths, transitions, head_transitions, tail_transitions):
    em = emissions[:, :, 0, :]
    tg = tags[:, :, 0]
    em_sc = jnp.take_along_axis(em, tg[..., None], axis=-1)[..., 0]
    tr_sc = transitions[0, 0][tg[:, :-1], tg[:, 1:]]
    head_sc = head_transitions[0, 0][tg[:, 0]]
    tail_tag = tg[jnp.arange(B), lengths - 1]
    tail_sc = tail_transitions[0, 0][tail_tag]
    mask = (jnp.arange(L)[None, :] < lengths[:, None]).astype(jnp.float32)
    mask_tr = (jnp.arange(1, L)[None, :] < lengths[:, None]).astype(jnp.float32)
    tot = jnp.sum(em_sc * mask, axis=1) + jnp.sum(tr_sc * mask_tr, axis=1)
    return (tot + head_sc + tail_sc)[:, None]


def kernel(emissions, tags, lengths, transitions, head_transitions, tail_transitions):
    em_t = jnp.transpose(emissions[:, :, 0, :], (1, 0, 2))
    trans = transitions[0, 0]
    head = head_transitions[0]
    tail = tail_transitions[0]
    lens2 = lengths[:, None].astype(jnp.int32)
    parts = _partitions_tc(em_t, trans, head, tail, lens2)
    sc_part = _scores_sc(emissions[:, :, 0, :].reshape(B, L * N),
                         lax.bitcast_convert_type(
                             tags[:, :, 0].astype(jnp.int32), jnp.float32),
                         lax.bitcast_convert_type(
                             lengths.astype(jnp.int32), jnp.float32),
                         trans.reshape(N * N),
                         head_transitions[0, 0], tail_transitions[0, 0])
    scores = jnp.sum(sc_part, axis=(1, 2))[:, None]
    return scores - parts


# speedup vs baseline: 1.0417x; 1.0417x over previous
import functools

import jax
import jax.numpy as jnp
from jax import lax
from jax.experimental import pallas as pl
from jax.experimental.pallas import tpu as pltpu
from jax.experimental.pallas import tpu_sc as plsc

B = 16
L = 2048
N = 64
CHUNK = 64
HALF = L // 2
NCHUNK = HALF // CHUNK
NORM_EVERY = 8


def _partition_body(emf_ref, emb_ref, trans_ref, head_ref, tail_ref, len_ref,
                    out_ref, x_ref, m_ref, mb_ref, pexp_ref, pm_ref,
                    w_ref, etail_ref, et128_ref, ef_ref, eb_ref):
    c = pl.program_id(0)
    lens = len_ref[...]
    lane = jax.lax.broadcasted_iota(jnp.int32, (1, 2 * N), 1)
    up_mask = lane >= N

    @pl.when(c == 0)
    def _init():
        tr = trans_ref[...]
        z = jnp.zeros((N, N), jnp.bfloat16)
        eT = jnp.exp(tr).astype(jnp.bfloat16)
        eTT = jnp.exp(tr.T).astype(jnp.bfloat16)
        w_ref[...] = jnp.concatenate(
            [jnp.concatenate([eT, z], axis=1),
             jnp.concatenate([z, eTT], axis=1)], axis=0)
        et = jnp.exp(tail_ref[...])
        etail_ref[...] = jnp.concatenate(
            [et, jnp.zeros((1, N), jnp.float32)], axis=1)
        et128 = jnp.concatenate([jnp.ones((1, N), jnp.float32), et], axis=1)
        et128_ref[...] = et128.astype(jnp.bfloat16)
        a0 = jnp.exp(emf_ref[0] + head_ref[...])
        x_ref[...] = jnp.concatenate(
            [a0, jnp.broadcast_to(et, (B, N))], axis=1).astype(jnp.bfloat16)
        m_ref[...] = jnp.zeros_like(m_ref)
        mb_ref[...] = jnp.zeros_like(mb_ref)
        pexp_ref[...] = jnp.ones_like(pexp_ref)
        pm_ref[...] = jnp.zeros_like(pm_ref)

    ef_ref[...] = jnp.exp(emf_ref[...]).astype(jnp.bfloat16)
    eb_ref[...] = jnp.exp(emb_ref[...]).astype(jnp.bfloat16)

    w = w_ref[...]
    capsel = etail_ref[...]
    et128 = et128_ref[...]
    x = x_ref[...]
    m = m_ref[...]
    mb = mb_ref[...]
    pexp = pexp_ref[...]
    pm = pm_ref[...]
    onesBN = jnp.ones((B, N), jnp.bfloat16)
    scale_vec = None
    log_sa = log_sb = None

    for s in range(CHUNK):
        i = c * CHUNK + s
        pre_s = jnp.concatenate([onesBN, eb_ref[CHUNK - 1 - s]], axis=1)
        post_s = jnp.concatenate(
            [ef_ref[s], onesBN], axis=1).astype(jnp.float32)
        if scale_vec is not None:
            post_s = post_s * scale_vec
        xp = x * pre_s
        upd = jax.lax.dot_general(
            xp, w, (((1,), (0,)), ((), ())),
            preferred_element_type=jnp.float32)
        upd = (upd * post_s).astype(jnp.bfloat16)
        if scale_vec is not None:
            m = m + log_sa
            mb = mb + log_sb
            scale_vec = None
        if s == 0:
            upd = jnp.where((i == 0) & jnp.logical_not(up_mask), x, upd)
        predb = lens == (L - 1 - i)
        x = jnp.where(predb & up_mask, et128, upd)
        mb = jnp.where(predb, 0.0, mb)
        xf = x.astype(jnp.float32)
        dotv = jnp.sum(xf * capsel, axis=1, keepdims=True)
        pred = lens == (i + 1)
        pexp = jnp.where(pred, dotv, pexp)
        pm = jnp.where(pred, m, pm)
        if s % NORM_EVERY == NORM_EVERY - 2:
            sa = jnp.max(jnp.where(up_mask, 0.0, xf), axis=1, keepdims=True)
            sb = jnp.max(jnp.where(up_mask, xf, 0.0), axis=1, keepdims=True)
            log_sa = jnp.log(sa)
            log_sb = jnp.log(sb)
            scale_vec = jnp.where(up_mask, 1.0 / sb, 1.0 / sa)

    x_ref[...] = x
    m_ref[...] = m
    mb_ref[...] = mb
    pexp_ref[...] = pexp
    pm_ref[...] = pm

    @pl.when(c == NCHUNK - 1)
    def _fin():
        xf = x_ref[...].astype(jnp.float32)
        prod = xf[:, :N] * xf[:, N:]
        bridge = jnp.sum(prod, axis=1, keepdims=True)
        plong = m_ref[...] + mb_ref[...] + jnp.log(bridge)
        pshort = pm_ref[...] + jnp.log(pexp_ref[...])
        out_ref[...] = jnp.where(lens > HALF, plong, pshort)


@functools.partial(jax.jit, static_argnames=("interpret",))
def _partitions_tc(em_t, trans, head, tail, lengths, interpret=False):
    nblk = L // CHUNK
    return pl.pallas_call(
        _partition_body,
        grid=(NCHUNK,),
        in_specs=[
            pl.BlockSpec((CHUNK, B, N), lambda c: (c, 0, 0)),
            pl.BlockSpec((CHUNK, B, N), lambda c: (nblk - 1 - c, 0, 0)),
            pl.BlockSpec((N, N), lambda c: (0, 0)),
            pl.BlockSpec((1, N), lambda c: (0, 0)),
            pl.BlockSpec((1, N), lambda c: (0, 0)),
            pl.BlockSpec((B, 1), lambda c: (0, 0)),
        ],
        out_specs=pl.BlockSpec((B, 1), lambda c: (0, 0)),
        out_shape=jax.ShapeDtypeStruct((B, 1), jnp.float32),
        scratch_shapes=[
            pltpu.VMEM((B, 2 * N), jnp.bfloat16),
            pltpu.VMEM((B, 1), jnp.float32),
            pltpu.VMEM((B, 1), jnp.float32),
            pltpu.VMEM((B, 1), jnp.float32),
            pltpu.VMEM((B, 1), jnp.float32),
            pltpu.VMEM((2 * N, 2 * N), jnp.bfloat16),
            pltpu.VMEM((1, 2 * N), jnp.float32),
            pltpu.VMEM((1, 2 * N), jnp.bfloat16),
            pltpu.VMEM((CHUNK, B, N), jnp.bfloat16),
            pltpu.VMEM((CHUNK, B, N), jnp.bfloat16),
        ],
        interpret=interpret,
    )(em_t, em_t, trans, head, tail, lengths)


HALF_L = L // 2


def _scores_sc_body(em_hbm, tags_hbm, lens_hbm, trans_hbm, head_hbm, tail_hbm,
                    out_hbm, em_v, tags_v, trans_v, ht_v, lens_v, out_v):
    wid = lax.axis_index("s") * 2 + lax.axis_index("c")
    b = wid // 2
    half = wid % 2
    t0 = half * HALF_L
    pltpu.sync_copy(em_hbm.at[b, pl.ds(t0 * N, HALF_L * N)], em_v)
    pltpu.sync_copy(tags_hbm.at[b], tags_v)
    pltpu.sync_copy(trans_hbm, trans_v)
    pltpu.sync_copy(head_hbm, ht_v.at[pl.ds(0, N)])
    pltpu.sync_copy(tail_hbm, ht_v.at[pl.ds(N, N)])
    pltpu.sync_copy(lens_hbm, lens_v.at[pl.ds(0, B)])

    lane = lax.iota(jnp.int32, 16)
    lenb = plsc.bitcast(
        plsc.load_gather(lens_v, [jnp.full((16,), b, jnp.int32)]), jnp.int32)
    zero = jnp.zeros((16,), jnp.float32)

    def body(i, carry):
        acc_em, acc_tr = carry
        tl = i * 16 + lane
        tg = tl + t0
        tagv = plsc.bitcast(plsc.load_gather(tags_v, [tg]), jnp.int32)
        prev = plsc.bitcast(
            plsc.load_gather(tags_v, [jnp.maximum(tg - 1, 0)]), jnp.int32)
        emv = plsc.load_gather(em_v, [tl * N + tagv])
        trv = plsc.load_gather(trans_v, [prev * N + tagv])
        m_em = tg < lenb
        m_tr = (tg >= 1) & (tg < lenb)
        acc_em = acc_em + jnp.where(m_em, emv, 0.0)
        acc_tr = acc_tr + jnp.where(m_tr, trv, 0.0)
        return acc_em, acc_tr

    acc_em, acc_tr = lax.fori_loop(0, HALF_L // 16, body, (zero, zero))
    total = acc_em + acc_tr

    halfv = jnp.full((16,), half, jnp.int32)
    lane0 = lane == 0
    tag0 = plsc.bitcast(
        plsc.load_gather(tags_v, [jnp.zeros((16,), jnp.int32)]), jnp.int32)
    headv = plsc.load_gather(ht_v, [tag0])
    total = total + jnp.where(lane0 & (halfv == 0), headv, 0.0)
    ttail = lenb - 1
    cover = (ttail >= t0) & (ttail < t0 + HALF_L)
    tagt = plsc.bitcast(plsc.load_gather(tags_v, [ttail]), jnp.int32)
    tailv = plsc.load_gather(ht_v, [N + tagt])
    total = total + jnp.where(lane0 & cover, tailv, 0.0)

    out_v[...] = total
    pltpu.sync_copy(out_v, out_hbm.at[b, half])


@jax.jit
def _scores_sc(em, tg, lens, trans, head, tail):
    k = pl.kernel(
        _scores_sc_body,
        out_type=jax.ShapeDtypeStruct((B, 2, 16), jnp.float32),
        mesh=plsc.VectorSubcoreMesh(core_axis_name="c", subcore_axis_name="s"),
        compiler_params=pltpu.CompilerParams(needs_layout_passes=False),
        scratch_types=[
            pltpu.VMEM((HALF_L * N,), jnp.float32),
            pltpu.VMEM((L,), jnp.float32),
            pltpu.VMEM((N * N,), jnp.float32),
            pltpu.VMEM((2 * N,), jnp.float32),
            pltpu.VMEM((128,), jnp.float32),
            pltpu.VMEM((16,), jnp.float32),
        ],
    )
    return k(em, tg, lens, trans, head, tail)


def _scores_jax(emissions, tags, lengths, transitions, head_transitions, tail_transitions):
    em = emissions[:, :, 0, :]
    tg = tags[:, :, 0]
    em_sc = jnp.take_along_axis(em, tg[..., None], axis=-1)[..., 0]
    tr_sc = transitions[0, 0][tg[:, :-1], tg[:, 1:]]
    head_sc = head_transitions[0, 0][tg[:, 0]]
    tail_tag = tg[jnp.arange(B), lengths - 1]
    tail_sc = tail_transitions[0, 0][tail_tag]
    mask = (jnp.arange(L)[None, :] < lengths[:, None]).astype(jnp.float32)
    mask_tr = (jnp.arange(1, L)[None, :] < lengths[:, None]).astype(jnp.float32)
    tot = jnp.sum(em_sc * mask, axis=1) + jnp.sum(tr_sc * mask_tr, axis=1)
    return (tot + head_sc + tail_sc)[:, None]


def kernel(emissions, tags, lengths, transitions, head_transitions, tail_transitions):
    em_t = jnp.transpose(emissions[:, :, 0, :], (1, 0, 2))
    trans = transitions[0, 0]
    head = head_transitions[0]
    tail = tail_transitions[0]
    lens2 = lengths[:, None].astype(jnp.int32)
    parts = _partitions_tc(em_t, trans, head, tail, lens2)
    sc_part = _scores_sc(emissions[:, :, 0, :].reshape(B, L * N),
                         lax.bitcast_convert_type(
                             tags[:, :, 0].astype(jnp.int32), jnp.float32),
                         lax.bitcast_convert_type(
                             lengths.astype(jnp.int32), jnp.float32),
                         trans.reshape(N * N),
                         head_transitions[0, 0], tail_transitions[0, 0])
    scores = jnp.sum(sc_part, axis=(1, 2))[:, None]
    return scores - parts


# speedup vs baseline: 1.0633x; 1.0207x over previous
import functools

import jax
import jax.numpy as jnp
from jax import lax
from jax.experimental import pallas as pl
from jax.experimental.pallas import tpu as pltpu
from jax.experimental.pallas import tpu_sc as plsc

B = 16
L = 2048
N = 64
CHUNK = 64
HALF = L // 2
NCHUNK = HALF // CHUNK
NORM_EVERY = 8


def _partition_body(emf_ref, emb_ref, trans_ref, head_ref, tail_ref, len_ref,
                    out_ref, x_ref, m_ref, mb_ref, pexp_ref, pm_ref,
                    w_ref, etail_ref, et128_ref, ef_ref, eb_ref):
    c = pl.program_id(0)
    lens = len_ref[...]
    lane = jax.lax.broadcasted_iota(jnp.int32, (1, 2 * N), 1)
    up_mask = lane >= N

    @pl.when(c == 0)
    def _init():
        tr = trans_ref[...]
        z = jnp.zeros((N, N), jnp.bfloat16)
        eT = jnp.exp(tr).astype(jnp.bfloat16)
        eTT = jnp.exp(tr.T).astype(jnp.bfloat16)
        w_ref[...] = jnp.concatenate(
            [jnp.concatenate([eT, z], axis=1),
             jnp.concatenate([z, eTT], axis=1)], axis=0)
        et = jnp.exp(tail_ref[...])
        etail_ref[...] = jnp.concatenate(
            [et, jnp.zeros((1, N), jnp.float32)], axis=1)
        et128 = jnp.concatenate([jnp.ones((1, N), jnp.float32), et], axis=1)
        et128_ref[...] = et128.astype(jnp.bfloat16)
        a0 = jnp.exp(emf_ref[:, 0, :] + head_ref[...])
        x_ref[...] = jnp.concatenate(
            [a0, jnp.broadcast_to(et, (B, N))], axis=1).astype(jnp.bfloat16)
        m_ref[...] = jnp.zeros_like(m_ref)
        mb_ref[...] = jnp.zeros_like(mb_ref)
        pexp_ref[...] = jnp.ones_like(pexp_ref)
        pm_ref[...] = jnp.zeros_like(pm_ref)

    ef_ref[...] = jnp.exp(
        jnp.transpose(emf_ref[...], (1, 0, 2))).astype(jnp.bfloat16)
    eb_ref[...] = jnp.exp(
        jnp.transpose(emb_ref[...], (1, 0, 2))).astype(jnp.bfloat16)

    w = w_ref[...]
    capsel = etail_ref[...]
    et128 = et128_ref[...]
    x = x_ref[...]
    m = m_ref[...]
    mb = mb_ref[...]
    pexp = pexp_ref[...]
    pm = pm_ref[...]
    onesBN = jnp.ones((B, N), jnp.bfloat16)
    scale_vec = None
    log_sa = log_sb = None

    for s in range(CHUNK):
        i = c * CHUNK + s
        pre_s = jnp.concatenate([onesBN, eb_ref[CHUNK - 1 - s]], axis=1)
        post_s = jnp.concatenate(
            [ef_ref[s], onesBN], axis=1).astype(jnp.float32)
        if scale_vec is not None:
            post_s = post_s * scale_vec
        xp = x * pre_s
        upd = jax.lax.dot_general(
            xp, w, (((1,), (0,)), ((), ())),
            preferred_element_type=jnp.float32)
        upd = (upd * post_s).astype(jnp.bfloat16)
        if scale_vec is not None:
            m = m + log_sa
            mb = mb + log_sb
            scale_vec = None
        if s == 0:
            upd = jnp.where((i == 0) & jnp.logical_not(up_mask), x, upd)
        predb = lens == (L - 1 - i)
        x = jnp.where(predb & up_mask, et128, upd)
        mb = jnp.where(predb, 0.0, mb)
        xf = x.astype(jnp.float32)
        dotv = jnp.sum(xf * capsel, axis=1, keepdims=True)
        pred = lens == (i + 1)
        pexp = jnp.where(pred, dotv, pexp)
        pm = jnp.where(pred, m, pm)
        if s % NORM_EVERY == NORM_EVERY - 2:
            sa = jnp.max(jnp.where(up_mask, 0.0, xf), axis=1, keepdims=True)
            sb = jnp.max(jnp.where(up_mask, xf, 0.0), axis=1, keepdims=True)
            log_sa = jnp.log(sa)
            log_sb = jnp.log(sb)
            scale_vec = jnp.where(up_mask, 1.0 / sb, 1.0 / sa)

    x_ref[...] = x
    m_ref[...] = m
    mb_ref[...] = mb
    pexp_ref[...] = pexp
    pm_ref[...] = pm

    @pl.when(c == NCHUNK - 1)
    def _fin():
        xf = x_ref[...].astype(jnp.float32)
        prod = xf[:, :N] * xf[:, N:]
        bridge = jnp.sum(prod, axis=1, keepdims=True)
        plong = m_ref[...] + mb_ref[...] + jnp.log(bridge)
        pshort = pm_ref[...] + jnp.log(pexp_ref[...])
        out_ref[...] = jnp.where(lens > HALF, plong, pshort)


@functools.partial(jax.jit, static_argnames=("interpret",))
def _partitions_tc(em3, trans, head, tail, lengths, interpret=False):
    nblk = L // CHUNK
    return pl.pallas_call(
        _partition_body,
        grid=(NCHUNK,),
        in_specs=[
            pl.BlockSpec((B, CHUNK, N), lambda c: (0, c, 0)),
            pl.BlockSpec((B, CHUNK, N), lambda c: (0, nblk - 1 - c, 0)),
            pl.BlockSpec((N, N), lambda c: (0, 0)),
            pl.BlockSpec((1, N), lambda c: (0, 0)),
            pl.BlockSpec((1, N), lambda c: (0, 0)),
            pl.BlockSpec((B, 1), lambda c: (0, 0)),
        ],
        out_specs=pl.BlockSpec((B, 1), lambda c: (0, 0)),
        out_shape=jax.ShapeDtypeStruct((B, 1), jnp.float32),
        scratch_shapes=[
            pltpu.VMEM((B, 2 * N), jnp.bfloat16),
            pltpu.VMEM((B, 1), jnp.float32),
            pltpu.VMEM((B, 1), jnp.float32),
            pltpu.VMEM((B, 1), jnp.float32),
            pltpu.VMEM((B, 1), jnp.float32),
            pltpu.VMEM((2 * N, 2 * N), jnp.bfloat16),
            pltpu.VMEM((1, 2 * N), jnp.float32),
            pltpu.VMEM((1, 2 * N), jnp.bfloat16),
            pltpu.VMEM((CHUNK, B, N), jnp.bfloat16),
            pltpu.VMEM((CHUNK, B, N), jnp.bfloat16),
        ],
        interpret=interpret,
    )(em3, em3, trans, head, tail, lengths)


HALF_L = L // 2


def _scores_sc_body(em_hbm, tags_hbm, lens_hbm, trans_hbm, head_hbm, tail_hbm,
                    out_hbm, em_v, tags_v, trans_v, ht_v, lens_v, out_v):
    wid = lax.axis_index("s") * 2 + lax.axis_index("c")
    b = wid // 2
    half = wid % 2
    t0 = half * HALF_L
    pltpu.sync_copy(em_hbm.at[b, pl.ds(t0 * N, HALF_L * N)], em_v)
    pltpu.sync_copy(tags_hbm.at[b], tags_v)
    pltpu.sync_copy(trans_hbm, trans_v)
    pltpu.sync_copy(head_hbm, ht_v.at[pl.ds(0, N)])
    pltpu.sync_copy(tail_hbm, ht_v.at[pl.ds(N, N)])
    pltpu.sync_copy(lens_hbm, lens_v.at[pl.ds(0, B)])

    lane = lax.iota(jnp.int32, 16)
    lenb = plsc.bitcast(
        plsc.load_gather(lens_v, [jnp.full((16,), b, jnp.int32)]), jnp.int32)
    zero = jnp.zeros((16,), jnp.float32)

    def body(i, carry):
        acc_em, acc_tr = carry
        tl = i * 16 + lane
        tg = tl + t0
        tagv = plsc.bitcast(plsc.load_gather(tags_v, [tg]), jnp.int32)
        prev = plsc.bitcast(
            plsc.load_gather(tags_v, [jnp.maximum(tg - 1, 0)]), jnp.int32)
        emv = plsc.load_gather(em_v, [tl * N + tagv])
        trv = plsc.load_gather(trans_v, [prev * N + tagv])
        m_em = tg < lenb
        m_tr = (tg >= 1) & (tg < lenb)
        acc_em = acc_em + jnp.where(m_em, emv, 0.0)
        acc_tr = acc_tr + jnp.where(m_tr, trv, 0.0)
        return acc_em, acc_tr

    acc_em, acc_tr = lax.fori_loop(0, HALF_L // 16, body, (zero, zero))
    total = acc_em + acc_tr

    halfv = jnp.full((16,), half, jnp.int32)
    lane0 = lane == 0
    tag0 = plsc.bitcast(
        plsc.load_gather(tags_v, [jnp.zeros((16,), jnp.int32)]), jnp.int32)
    headv = plsc.load_gather(ht_v, [tag0])
    total = total + jnp.where(lane0 & (halfv == 0), headv, 0.0)
    ttail = lenb - 1
    cover = (ttail >= t0) & (ttail < t0 + HALF_L)
    tagt = plsc.bitcast(plsc.load_gather(tags_v, [ttail]), jnp.int32)
    tailv = plsc.load_gather(ht_v, [N + tagt])
    total = total + jnp.where(lane0 & cover, tailv, 0.0)

    out_v[...] = total
    pltpu.sync_copy(out_v, out_hbm.at[b, half])


@jax.jit
def _scores_sc(em, tg, lens, trans, head, tail):
    k = pl.kernel(
        _scores_sc_body,
        out_type=jax.ShapeDtypeStruct((B, 2, 16), jnp.float32),
        mesh=plsc.VectorSubcoreMesh(core_axis_name="c", subcore_axis_name="s"),
        compiler_params=pltpu.CompilerParams(needs_layout_passes=False),
        scratch_types=[
            pltpu.VMEM((HALF_L * N,), jnp.float32),
            pltpu.VMEM((L,), jnp.float32),
            pltpu.VMEM((N * N,), jnp.float32),
            pltpu.VMEM((2 * N,), jnp.float32),
            pltpu.VMEM((128,), jnp.float32),
            pltpu.VMEM((16,), jnp.float32),
        ],
    )
    return k(em, tg, lens, trans, head, tail)


def _scores_jax(emissions, tags, lengths, transitions, head_transitions, tail_transitions):
    em = emissions[:, :, 0, :]
    tg = tags[:, :, 0]
    em_sc = jnp.take_along_axis(em, tg[..., None], axis=-1)[..., 0]
    tr_sc = transitions[0, 0][tg[:, :-1], tg[:, 1:]]
    head_sc = head_transitions[0, 0][tg[:, 0]]
    tail_tag = tg[jnp.arange(B), lengths - 1]
    tail_sc = tail_transitions[0, 0][tail_tag]
    mask = (jnp.arange(L)[None, :] < lengths[:, None]).astype(jnp.float32)
    mask_tr = (jnp.arange(1, L)[None, :] < lengths[:, None]).astype(jnp.float32)
    tot = jnp.sum(em_sc * mask, axis=1) + jnp.sum(tr_sc * mask_tr, axis=1)
    return (tot + head_sc + tail_sc)[:, None]


def kernel(emissions, tags, lengths, transitions, head_transitions, tail_transitions):
    em3 = emissions[:, :, 0, :]
    trans = transitions[0, 0]
    head = head_transitions[0]
    tail = tail_transitions[0]
    lens2 = lengths[:, None].astype(jnp.int32)
    parts = _partitions_tc(em3, trans, head, tail, lens2)
    sc_part = _scores_sc(emissions[:, :, 0, :].reshape(B, L * N),
                         lax.bitcast_convert_type(
                             tags[:, :, 0].astype(jnp.int32), jnp.float32),
                         lax.bitcast_convert_type(
                             lengths.astype(jnp.int32), jnp.float32),
                         trans.reshape(N * N),
                         head_transitions[0, 0], tail_transitions[0, 0])
    scores = jnp.sum(sc_part, axis=(1, 2))[:, None]
    return scores - parts


# speedup vs baseline: 1.0649x; 1.0015x over previous
import functools

import jax
import jax.numpy as jnp
from jax import lax
from jax.experimental import pallas as pl
from jax.experimental.pallas import tpu as pltpu
from jax.experimental.pallas import tpu_sc as plsc

B = 16
L = 2048
N = 64
CHUNK = 64
HALF = L // 2
NCHUNK = HALF // CHUNK
NORM_EVERY = 8


def _partition_body(emf_ref, emb_ref, trans_ref, head_ref, tail_ref, len_ref,
                    out_ref, x_ref, m_ref, mb_ref, pexp_ref, pm_ref,
                    w_ref, etail_ref, et128_ref, ef_ref, eb_ref):
    c = pl.program_id(0)
    lens = len_ref[...]
    lane = jax.lax.broadcasted_iota(jnp.int32, (1, 2 * N), 1)
    up_mask = lane >= N

    @pl.when(c == 0)
    def _init():
        tr = trans_ref[...]
        z = jnp.zeros((N, N), jnp.bfloat16)
        eT = jnp.exp(tr).astype(jnp.bfloat16)
        eTT = jnp.exp(tr.T).astype(jnp.bfloat16)
        w_ref[...] = jnp.concatenate(
            [jnp.concatenate([eT, z], axis=1),
             jnp.concatenate([z, eTT], axis=1)], axis=0)
        et = jnp.exp(tail_ref[...])
        etail_ref[...] = jnp.concatenate(
            [et, jnp.zeros((1, N), jnp.float32)], axis=1)
        et128 = jnp.concatenate([jnp.ones((1, N), jnp.float32), et], axis=1)
        et128_ref[...] = et128.astype(jnp.bfloat16)
        a0 = jnp.exp(emf_ref[:, 0, :] + head_ref[...])
        x_ref[...] = jnp.concatenate(
            [a0, jnp.broadcast_to(et, (B, N))], axis=1).astype(jnp.bfloat16)
        m_ref[...] = jnp.zeros_like(m_ref)
        mb_ref[...] = jnp.zeros_like(mb_ref)
        pexp_ref[...] = jnp.ones_like(pexp_ref)
        pm_ref[...] = jnp.zeros_like(pm_ref)

    ef_ref[...] = jnp.exp(
        jnp.transpose(emf_ref[...], (1, 0, 2))).astype(jnp.bfloat16)
    eb_ref[...] = jnp.exp(
        jnp.transpose(emb_ref[...], (1, 0, 2))).astype(jnp.bfloat16)

    w = w_ref[...]
    capsel = etail_ref[...]
    et128 = et128_ref[...]
    x = x_ref[...]
    m = m_ref[...]
    mb = mb_ref[...]
    pexp = pexp_ref[...]
    pm = pm_ref[...]
    onesBN = jnp.ones((B, N), jnp.bfloat16)
    scale_vec = None
    log_sa = log_sb = None

    for s in range(CHUNK):
        i = c * CHUNK + s
        pre_s = jnp.concatenate([onesBN, eb_ref[CHUNK - 1 - s]], axis=1)
        post_s = jnp.concatenate(
            [ef_ref[s], onesBN], axis=1).astype(jnp.float32)
        if scale_vec is not None:
            post_s = post_s * scale_vec
        xp = x * pre_s
        upd = jax.lax.dot_general(
            xp, w, (((1,), (0,)), ((), ())),
            preferred_element_type=jnp.float32)
        upd = (upd * post_s).astype(jnp.bfloat16)
        if scale_vec is not None:
            m = m + log_sa
            mb = mb + log_sb
            scale_vec = None
        if s == 0:
            upd = jnp.where((i == 0) & jnp.logical_not(up_mask), x, upd)
        predb = lens == (L - 1 - i)
        x = jnp.where(predb & up_mask, et128, upd)
        mb = jnp.where(predb, 0.0, mb)
        xf = x.astype(jnp.float32)
        dotv = jnp.sum(xf * capsel, axis=1, keepdims=True)
        pred = lens == (i + 1)
        pexp = jnp.where(pred, dotv, pexp)
        pm = jnp.where(pred, m, pm)
        if s % NORM_EVERY == NORM_EVERY - 2:
            sa = jnp.max(jnp.where(up_mask, 0.0, xf), axis=1, keepdims=True)
            sb = jnp.max(jnp.where(up_mask, xf, 0.0), axis=1, keepdims=True)
            log_sa = jnp.log(sa)
            log_sb = jnp.log(sb)
            scale_vec = jnp.where(up_mask, 1.0 / sb, 1.0 / sa)

    x_ref[...] = x
    m_ref[...] = m
    mb_ref[...] = mb
    pexp_ref[...] = pexp
    pm_ref[...] = pm

    @pl.when(c == NCHUNK - 1)
    def _fin():
        xf = x_ref[...].astype(jnp.float32)
        prod = xf[:, :N] * xf[:, N:]
        bridge = jnp.sum(prod, axis=1, keepdims=True)
        plong = m_ref[...] + mb_ref[...] + jnp.log(bridge)
        pshort = pm_ref[...] + jnp.log(pexp_ref[...])
        out_ref[...] = jnp.where(lens > HALF, plong, pshort)


@functools.partial(jax.jit, static_argnames=("interpret",))
def _partitions_tc(em3, trans, head, tail, lengths, interpret=False):
    nblk = L // CHUNK
    return pl.pallas_call(
        _partition_body,
        grid=(NCHUNK,),
        in_specs=[
            pl.BlockSpec((B, CHUNK, N), lambda c: (0, c, 0)),
            pl.BlockSpec((B, CHUNK, N), lambda c: (0, nblk - 1 - c, 0)),
            pl.BlockSpec((N, N), lambda c: (0, 0)),
            pl.BlockSpec((1, N), lambda c: (0, 0)),
            pl.BlockSpec((1, N), lambda c: (0, 0)),
            pl.BlockSpec((B, 1), lambda c: (0, 0)),
        ],
        out_specs=pl.BlockSpec((B, 1), lambda c: (0, 0)),
        out_shape=jax.ShapeDtypeStruct((B, 1), jnp.float32),
        scratch_shapes=[
            pltpu.VMEM((B, 2 * N), jnp.bfloat16),
            pltpu.VMEM((B, 1), jnp.float32),
            pltpu.VMEM((B, 1), jnp.float32),
            pltpu.VMEM((B, 1), jnp.float32),
            pltpu.VMEM((B, 1), jnp.float32),
            pltpu.VMEM((2 * N, 2 * N), jnp.bfloat16),
            pltpu.VMEM((1, 2 * N), jnp.float32),
            pltpu.VMEM((1, 2 * N), jnp.bfloat16),
            pltpu.VMEM((CHUNK, B, N), jnp.bfloat16),
            pltpu.VMEM((CHUNK, B, N), jnp.bfloat16),
        ],
        interpret=interpret,
    )(em3, em3, trans, head, tail, lengths)


HALF_L = L // 2


def _scores_sc_body(em_hbm, tags_hbm, lens_hbm, trans_hbm, head_hbm, tail_hbm,
                    out_hbm, em_v, tags_v, trans_v, ht_v, lens_v, out_v):
    wid = lax.axis_index("s") * 2 + lax.axis_index("c")
    b = wid // 2
    half = wid % 2
    t0 = half * HALF_L
    pltpu.sync_copy(em_hbm.at[b, pl.ds(t0 * N, HALF_L * N)], em_v)
    pltpu.sync_copy(tags_hbm.at[b], tags_v)
    pltpu.sync_copy(trans_hbm, trans_v)
    pltpu.sync_copy(head_hbm, ht_v.at[pl.ds(0, N)])
    pltpu.sync_copy(tail_hbm, ht_v.at[pl.ds(N, N)])
    pltpu.sync_copy(lens_hbm, lens_v.at[pl.ds(0, B)])

    lane = lax.iota(jnp.int32, 16)
    lenb = plsc.bitcast(
        plsc.load_gather(lens_v, [jnp.full((16,), b, jnp.int32)]), jnp.int32)
    zero = jnp.zeros((16,), jnp.float32)

    def body(i, carry):
        acc_em, acc_tr = carry
        tl = i * 16 + lane
        tg = tl + t0
        tagv = plsc.bitcast(plsc.load_gather(tags_v, [tg]), jnp.int32)
        prev = plsc.bitcast(
            plsc.load_gather(tags_v, [jnp.maximum(tg - 1, 0)]), jnp.int32)
        emv = plsc.load_gather(em_v, [tl * N + tagv])
        trv = plsc.load_gather(trans_v, [prev * N + tagv])
        m_em = tg < lenb
        m_tr = (tg >= 1) & (tg < lenb)
        acc_em = acc_em + jnp.where(m_em, emv, 0.0)
        acc_tr = acc_tr + jnp.where(m_tr, trv, 0.0)
        return acc_em, acc_tr

    acc_em, acc_tr = lax.fori_loop(0, HALF_L // 16, body, (zero, zero))
    total = acc_em + acc_tr

    halfv = jnp.full((16,), half, jnp.int32)
    lane0 = lane == 0
    tag0 = plsc.bitcast(
        plsc.load_gather(tags_v, [jnp.zeros((16,), jnp.int32)]), jnp.int32)
    headv = plsc.load_gather(ht_v, [tag0])
    total = total + jnp.where(lane0 & (halfv == 0), headv, 0.0)
    ttail = lenb - 1
    cover = (ttail >= t0) & (ttail < t0 + HALF_L)
    tagt = plsc.bitcast(plsc.load_gather(tags_v, [ttail]), jnp.int32)
    tailv = plsc.load_gather(ht_v, [N + tagt])
    total = total + jnp.where(lane0 & cover, tailv, 0.0)

    out_v[...] = total
    pltpu.sync_copy(out_v, out_hbm.at[b, half])


@jax.jit
def _scores_sc(em, tg, lens, trans, head, tail):
    k = pl.kernel(
        _scores_sc_body,
        out_type=jax.ShapeDtypeStruct((B, 2, 16), jnp.float32),
        mesh=plsc.VectorSubcoreMesh(core_axis_name="c", subcore_axis_name="s"),
        compiler_params=pltpu.CompilerParams(needs_layout_passes=False),
        scratch_types=[
            pltpu.VMEM((HALF_L * N,), jnp.float32),
            pltpu.VMEM((L,), jnp.float32),
            pltpu.VMEM((N * N,), jnp.float32),
            pltpu.VMEM((2 * N,), jnp.float32),
            pltpu.VMEM((128,), jnp.float32),
            pltpu.VMEM((16,), jnp.float32),
        ],
    )
    return k(em, tg, lens, trans, head, tail)


def _scores_jax(emissions, tags, lengths, transitions, head_transitions, tail_transitions):
    em = emissions[:, :, 0, :]
    tg = tags[:, :, 0]
    em_sc = jnp.take_along_axis(em, tg[..., None], axis=-1)[..., 0]
    tr_sc = transitions[0, 0][tg[:, :-1], tg[:, 1:]]
    head_sc = head_transitions[0, 0][tg[:, 0]]
    tail_tag = tg[jnp.arange(B), lengths - 1]
    tail_sc = tail_transitions[0, 0][tail_tag]
    mask = (jnp.arange(L)[None, :] < lengths[:, None]).astype(jnp.float32)
    mask_tr = (jnp.arange(1, L)[None, :] < lengths[:, None]).astype(jnp.float32)
    tot = jnp.sum(em_sc * mask, axis=1) + jnp.sum(tr_sc * mask_tr, axis=1)
    return (tot + head_sc + tail_sc)[:, None]


def kernel(emissions, tags, lengths, transitions, head_transitions, tail_transitions):
    em3 = emissions[:, :, 0, :]
    trans = transitions[0, 0]
    head = head_transitions[0]
    tail = tail_transitions[0]
    lens2 = lengths[:, None].astype(jnp.int32)
    sc_part = _scores_sc(em3.reshape(B, L * N),
                         lax.bitcast_convert_type(
                             tags[:, :, 0].astype(jnp.int32), jnp.float32),
                         lax.bitcast_convert_type(
                             lengths.astype(jnp.int32), jnp.float32),
                         trans.reshape(N * N),
                         head_transitions[0, 0], tail_transitions[0, 0])
    parts = _partitions_tc(em3, trans, head, tail, lens2)
    scores = jnp.sum(sc_part, axis=(1, 2))[:, None]
    return scores - parts
